# baseline (speedup 1.0000x reference)
import functools

import jax
import jax.numpy as jnp
from jax import lax
from jax.experimental import pallas as pl
from jax.experimental.pallas import tpu as pltpu
from jax.experimental.pallas import tpu_sc as plsc

VOCAB = 100000
D = 128
C = 128
B = 4096
L = 200

NC = 2
NS = 16
NW = NC * NS
RPW = B // NW
CHUNK = 100
CPR = L // CHUNK
NCHUNK = RPW * CPR
NVR = D // 16


def _accum_chunk(buf, acc):
  def body(j, a):
    return tuple(a[k] + buf[j, pl.ds(16 * k, 16)] for k in range(NVR))
  return lax.fori_loop(0, CHUNK, body, acc, unroll=10)


def _sc_body(table, idxh, outh, idx_v, buf0, buf1, out_v, sem0, sem1):
  cid = lax.axis_index("c")
  sid = lax.axis_index("s")
  wid = sid * NC + cid
  pltpu.sync_copy(idxh.at[pl.ds(wid * NCHUNK, NCHUNK)], idx_v)
  pltpu.async_copy(table.at[idx_v.at[0]], buf0, sem0)
  pltpu.async_copy(table.at[idx_v.at[1]], buf1, sem1)

  zeros = tuple(jnp.zeros((16,), jnp.float32) for _ in range(NVR))
  inv_l = jnp.float32(1.0 / L)

  def row_body(r, carry):
    cc = 2 * r
    pltpu.make_async_copy(table.at[idx_v.at[0]], buf0, sem0).wait()
    acc = _accum_chunk(buf0, zeros)
    nxt0 = jnp.minimum(cc + 2, NCHUNK - 2)
    pltpu.async_copy(table.at[idx_v.at[nxt0]], buf0, sem0)
    pltpu.make_async_copy(table.at[idx_v.at[1]], buf1, sem1).wait()
    acc = _accum_chunk(buf1, acc)
    nxt1 = jnp.minimum(cc + 3, NCHUNK - 1)
    pltpu.async_copy(table.at[idx_v.at[nxt1]], buf1, sem1)
    for k in range(NVR):
      out_v[r, pl.ds(16 * k, 16)] = acc[k] * inv_l
    return carry

  lax.fori_loop(0, RPW, row_body, 0)
  pltpu.make_async_copy(table.at[idx_v.at[0]], buf0, sem0).wait()
  pltpu.make_async_copy(table.at[idx_v.at[1]], buf1, sem1).wait()
  pltpu.sync_copy(out_v, outh.at[pl.ds(wid * RPW, RPW)])


def _sc_pool(table, idx2d):
  mesh = plsc.VectorSubcoreMesh(core_axis_name="c", subcore_axis_name="s")
  return pl.kernel(
      _sc_body,
      out_type=jax.ShapeDtypeStruct((B, D), jnp.float32),
      mesh=mesh,
      scratch_types=[
          pltpu.VMEM((NCHUNK, CHUNK), jnp.int32),
          pltpu.VMEM((CHUNK, D), jnp.float32),
          pltpu.VMEM((CHUNK, D), jnp.float32),
          pltpu.VMEM((RPW, D), jnp.float32),
          pltpu.SemaphoreType.DMA,
          pltpu.SemaphoreType.DMA,
      ],
  )(table, idx2d)


def _head_body(x_ref, w_ref, b_ref, g_ref, be_ref, lab_ref,
               loss_ref, preds_ref):
  x = x_ref[...]
  w = w_ref[...]
  y = lax.dot_general(x, w, (((1,), (1,)), ((), ())),
                      preferred_element_type=jnp.float32) + b_ref[...]
  mu = jnp.mean(y, axis=-1, keepdims=True)
  d = y - mu
  var = jnp.mean(d * d, axis=-1, keepdims=True)
  xn = d * lax.rsqrt(var + 1e-5) * g_ref[...] + be_ref[...]
  m = jnp.max(xn, axis=-1, keepdims=True)
  e = jnp.exp(xn - m)
  lse = jnp.log(jnp.sum(e, axis=-1, keepdims=True)) + m
  col = lax.broadcasted_iota(jnp.int32, (B, C), 1)
  picked = jnp.sum(jnp.where(col == lab_ref[...], xn, 0.0),
                   axis=-1, keepdims=True)
  loss_ref[...] = jnp.sum(lse - picked, axis=(0, 1), keepdims=True) * (1.0 / B)
  preds_ref[...] = jnp.argmax(xn, axis=-1).astype(jnp.int32)[:, None]


def _head(pooled, label_ids, W, b, gamma, beta):
  return pl.pallas_call(
      _head_body,
      out_shape=(
          jax.ShapeDtypeStruct((1, 1), jnp.float32),
          jax.ShapeDtypeStruct((B, 1), jnp.int32),
      ),
  )(pooled, W, b.reshape(1, C), gamma.reshape(1, C), beta.reshape(1, C),
    label_ids.reshape(B, 1).astype(jnp.int32))


@jax.jit
def kernel(word_ids, label_ids, embed_table, W, b, gamma, beta):
  idx2d = word_ids.astype(jnp.int32).reshape(B * CPR, CHUNK)
  pooled = _sc_pool(embed_table, idx2d)
  loss, preds = _head(pooled, label_ids, W, b, gamma, beta)
  return loss[0, 0], preds[:, 0]


# speedup vs baseline: 1.4399x; 1.4399x over previous
import functools

import jax
import jax.numpy as jnp
from jax import lax
from jax.experimental import pallas as pl
from jax.experimental.pallas import tpu as pltpu
from jax.experimental.pallas import tpu_sc as plsc

VOCAB = 100000
D = 128
C = 128
B = 4096
L = 200

NC = 2
NS = 16
NW = NC * NS
RPW = B // NW
CHUNK = 100
CPR = L // CHUNK
NCHUNK = RPW * CPR
NVR = D // 16


def _accum_chunk(buf, acc):
  def body(j, a):
    return tuple(a[k] + buf[j, pl.ds(16 * k, 16)] for k in range(NVR))
  return lax.fori_loop(0, CHUNK, body, acc, unroll=10)


NBUF = 4


def _sc_body(table, idxh, outh, idx_v, bufs, out_v, sems):
  cid = lax.axis_index("c")
  sid = lax.axis_index("s")
  wid = sid * NC + cid
  pltpu.sync_copy(idxh.at[pl.ds(wid * NCHUNK, NCHUNK)], idx_v)
  for bb in range(NBUF):
    pltpu.async_copy(table.at[idx_v.at[bb]], bufs[bb], sems[bb])

  zeros = tuple(jnp.zeros((16,), jnp.float32) for _ in range(NVR))
  inv_l = jnp.float32(1.0 / L)

  def iter_body(i, carry):
    cc = NBUF * i
    for bb in range(NBUF):
      r = (cc + bb) // CPR
      pltpu.make_async_copy(table.at[idx_v.at[bb]], bufs[bb], sems[bb]).wait()
      acc = _accum_chunk(bufs[bb], zeros if bb % CPR == 0 else acc)
      nxt = jnp.minimum(cc + bb + NBUF, NCHUNK - NBUF + bb)
      pltpu.async_copy(table.at[idx_v.at[nxt]], bufs[bb], sems[bb])
      if bb % CPR == CPR - 1:
        for k in range(NVR):
          out_v[r, pl.ds(16 * k, 16)] = acc[k] * inv_l
    return carry

  lax.fori_loop(0, NCHUNK // NBUF, iter_body, 0)
  for bb in range(NBUF):
    pltpu.make_async_copy(table.at[idx_v.at[bb]], bufs[bb], sems[bb]).wait()
  pltpu.sync_copy(out_v, outh.at[pl.ds(wid * RPW, RPW)])


def _sc_pool(table, idx2d):
  mesh = plsc.VectorSubcoreMesh(core_axis_name="c", subcore_axis_name="s")
  return pl.kernel(
      _sc_body,
      out_type=jax.ShapeDtypeStruct((B, D), jnp.float32),
      mesh=mesh,
      scratch_types=[
          pltpu.VMEM((NCHUNK, CHUNK), jnp.int32),
          [pltpu.VMEM((CHUNK, D), jnp.float32) for _ in range(NBUF)],
          pltpu.VMEM((RPW, D), jnp.float32),
          [pltpu.SemaphoreType.DMA for _ in range(NBUF)],
      ],
  )(table, idx2d)


def _head_body(x_ref, w_ref, b_ref, g_ref, be_ref, lab_ref,
               loss_ref, preds_ref):
  x = x_ref[...]
  w = w_ref[...]
  y = lax.dot_general(x, w, (((1,), (1,)), ((), ())),
                      preferred_element_type=jnp.float32) + b_ref[...]
  mu = jnp.mean(y, axis=-1, keepdims=True)
  d = y - mu
  var = jnp.mean(d * d, axis=-1, keepdims=True)
  xn = d * lax.rsqrt(var + 1e-5) * g_ref[...] + be_ref[...]
  m = jnp.max(xn, axis=-1, keepdims=True)
  e = jnp.exp(xn - m)
  lse = jnp.log(jnp.sum(e, axis=-1, keepdims=True)) + m
  col = lax.broadcasted_iota(jnp.int32, (B, C), 1)
  picked = jnp.sum(jnp.where(col == lab_ref[...], xn, 0.0),
                   axis=-1, keepdims=True)
  loss_ref[...] = jnp.sum(lse - picked, axis=(0, 1), keepdims=True) * (1.0 / B)
  preds_ref[...] = jnp.argmax(xn, axis=-1).astype(jnp.int32)[:, None]


def _head(pooled, label_ids, W, b, gamma, beta):
  return pl.pallas_call(
      _head_body,
      out_shape=(
          jax.ShapeDtypeStruct((1, 1), jnp.float32),
          jax.ShapeDtypeStruct((B, 1), jnp.int32),
      ),
  )(pooled, W, b.reshape(1, C), gamma.reshape(1, C), beta.reshape(1, C),
    label_ids.reshape(B, 1).astype(jnp.int32))


@jax.jit
def kernel(word_ids, label_ids, embed_table, W, b, gamma, beta):
  idx2d = word_ids.astype(jnp.int32).reshape(B * CPR, CHUNK)
  pooled = _sc_pool(embed_table, idx2d)
  loss, preds = _head(pooled, label_ids, W, b, gamma, beta)
  return loss[0, 0], preds[:, 0]
